# Initial kernel scaffold; baseline (speedup 1.0000x reference)
import jax
import jax.numpy as jnp
from jax.experimental import pallas as pl


def kernel(input):
    raise NotImplementedError("write your pallas kernel here")


# speedup vs baseline: 5.0918x; 5.0918x over previous
import functools

import jax
import jax.numpy as jnp
from jax.experimental import pallas as pl
from jax.experimental.pallas import tpu as pltpu

_KFRAC = 0.2
_ALPHA = 0.7
_ITERS = 16
_ROWS_PER_BLOCK = 256


def _kth_largest_approx(x, k):
    lo = jnp.min(x, axis=1, keepdims=True)
    hi = jnp.max(x, axis=1, keepdims=True)

    def body(_, carry):
        lo, hi = carry
        mid = 0.5 * (lo + hi)
        cnt = jnp.sum((x > mid).astype(jnp.float32), axis=1, keepdims=True)
        ge = cnt >= k
        return jnp.where(ge, mid, lo), jnp.where(ge, hi, mid)

    lo, hi = jax.lax.fori_loop(0, _ITERS, body, (lo, hi))
    return lo


def _pool_body(x_ref, o_ref, *, kmax, kmin):
    x = x_ref[...]
    r = x.shape[0]

    t1 = _kth_largest_approx(x, kmax)
    sum_top = kmax * t1[:, 0] + jnp.sum(
        jnp.maximum(x - t1, 0.0), axis=1)

    neg = -x
    t2 = _kth_largest_approx(neg, kmin)
    sum_bot_neg = kmin * t2[:, 0] + jnp.sum(
        jnp.maximum(neg - t2, 0.0), axis=1)

    out = sum_top / kmax - (_ALPHA / kmin) * sum_bot_neg
    o_ref[...] = out.reshape(1, r)


def kernel(input):
    b, c, h, w = input.shape
    n = h * w
    kmax = round(_KFRAC * n)
    kmin = round(_KFRAC * n)
    rows = b * c
    rpb = _ROWS_PER_BLOCK
    assert rows % rpb == 0
    nblocks = rows // rpb

    flat = input.reshape(rows, n)
    out = pl.pallas_call(
        functools.partial(_pool_body, kmax=kmax, kmin=kmin),
        grid=(nblocks,),
        in_specs=[pl.BlockSpec((rpb, n), lambda i: (i, 0))],
        out_specs=pl.BlockSpec((1, rpb), lambda i: (0, i)),
        out_shape=jax.ShapeDtypeStruct((1, rows), jnp.float32),
    )(flat)
    return out.reshape(b, c)
